# baseline (speedup 1.0000x reference)
import functools

import jax
import jax.numpy as jnp
from jax import lax
from jax.experimental import pallas as pl
from jax.experimental.pallas import tpu as pltpu
from jax.experimental.pallas import tpu_sc as plsc

N_NODES = 10000
N_EDGES = 320000
D = 128
DH = D // 2
NUM_RELS = 8

NC = 2
NS = 16
EPS = N_EDGES // NS
CH = 128
STEPS = 160
PADE = STEPS * CH - EPS
NPAD = 10240
ZR = NPAD // NS


def _node_transform_body(h_ref, op_ref, norm_ref, w_ref, t_ref):
    h = h_ref[...]
    op = op_ref[...]
    norm = norm_ref[...]
    acc = jnp.zeros((N_NODES, D), jnp.float32)
    for r in range(NUM_RELS):
        scale = jnp.where(op == r, norm, 0.0)
        acc += jnp.dot(h * scale, w_ref[r], preferred_element_type=jnp.float32)
    t_ref[0] = acc[:, :DH]
    t_ref[1] = acc[:, DH:]


def _node_transform(h, op2, norm2, weight):
    return pl.pallas_call(
        _node_transform_body,
        out_shape=jax.ShapeDtypeStruct((NC, N_NODES, DH), jnp.float32),
    )(h, op2, norm2, weight)


_sc_mesh = plsc.VectorSubcoreMesh(
    core_axis_name="c", subcore_axis_name="s", num_cores=NC, num_subcores=NS
)


@functools.partial(
    pl.kernel,
    out_type=jax.ShapeDtypeStruct((NC, NPAD, DH), jnp.float32),
    mesh=_sc_mesh,
    compiler_params=pltpu.CompilerParams(use_tc_tiling_on_sc=False),
    scratch_types=[
        pltpu.VMEM((STEPS, CH), jnp.int32),
        pltpu.VMEM((STEPS, CH), jnp.int32),
        pltpu.VMEM((4, CH, DH), jnp.float32),
        pltpu.VMEM_SHARED((NPAD, DH), jnp.float32),
        pltpu.SemaphoreType.DMA,
        pltpu.SemaphoreType.DMA,
        pltpu.SemaphoreType.DMA,
        pltpu.SemaphoreType.DMA,
        pltpu.SemaphoreType.DMA,
        pltpu.SemaphoreType.DMA,
        pltpu.SemaphoreType.DMA,
        pltpu.SemaphoreType.DMA,
    ],
)
def _sc_scatter(t_hbm, src_hbm, dst_hbm, zeros_hbm, out_hbm,
                src_v, dst_v, rows_v, acc,
                g0, g1, g2, g3, s0, s1, s2, s3):
    cid = lax.axis_index("c")
    sid = lax.axis_index("s")
    gsem = (g0, g1, g2, g3)
    ssem = (s0, s1, s2, s3)
    tc = t_hbm.at[cid]

    pltpu.sync_copy(zeros_hbm, acc.at[pl.ds(sid * ZR, ZR)])
    pltpu.sync_copy(src_hbm.at[sid], src_v)
    pltpu.sync_copy(dst_hbm.at[sid], dst_v)
    plsc.subcore_barrier()

    def fire_gather(s, b):
        pltpu.async_copy(tc.at[src_v.at[s]], rows_v.at[b], gsem[b])

    def wait_gather(s, b):
        pltpu.make_async_copy(tc.at[src_v.at[s]], rows_v.at[b],
                              gsem[b]).wait()

    def fire_scatter(s, b):
        pltpu.async_copy(rows_v.at[b], acc.at[dst_v.at[s]], ssem[b], add=True)

    def wait_scatter(s, b):
        pltpu.make_async_copy(rows_v.at[b], acc.at[dst_v.at[s]],
                              ssem[b]).wait()

    fire_gather(0, 0)
    fire_gather(1, 1)
    wait_gather(0, 0)
    fire_scatter(0, 0)
    fire_gather(2, 2)
    wait_gather(1, 1)
    fire_scatter(1, 1)
    fire_gather(3, 3)

    @pl.loop(2, STEPS - 2, step=4)
    def _slots(s):
        for j in range(4):
            b = (2 + j) % 4
            wait_gather(s + j, b)
            fire_scatter(s + j, b)
            wait_scatter(s + j - 2, (b + 2) % 4)
            fire_gather(s + j + 2, (b + 2) % 4)

    for j in range(2):
        s = STEPS - 2 + j
        b = (2 + j) % 4
        wait_gather(s, b)
        fire_scatter(s, b)
        wait_scatter(s - 2, (b + 2) % 4)
    wait_scatter(STEPS - 2, 2)
    wait_scatter(STEPS - 1, 3)

    plsc.subcore_barrier()
    pltpu.sync_copy(acc.at[pl.ds(sid * ZR, ZR)],
                    out_hbm.at[cid, pl.ds(sid * ZR, ZR)])


def _merge_body(p_ref, o_ref):
    o_ref[...] = jnp.concatenate(
        [p_ref[0, :N_NODES, :], p_ref[1, :N_NODES, :]], axis=1)


def _merge(partials):
    return pl.pallas_call(
        _merge_body,
        out_shape=jax.ShapeDtypeStruct((N_NODES, D), jnp.float32),
    )(partials)


def kernel(h, edge_index, op_class_id, norm, weight):
    src = edge_index[0].astype(jnp.int32).reshape(NS, EPS)
    dst = edge_index[1].astype(jnp.int32).reshape(NS, EPS)
    src = jnp.concatenate(
        [src, jnp.zeros((NS, PADE), jnp.int32)], axis=1).reshape(NS, STEPS, CH)
    dst = jnp.concatenate(
        [dst, jnp.full((NS, PADE), NPAD - 1, jnp.int32)], axis=1
    ).reshape(NS, STEPS, CH)
    op2 = op_class_id.astype(jnp.int32).reshape(N_NODES, 1)
    norm2 = norm.astype(jnp.float32).reshape(N_NODES, 1)
    t = _node_transform(h, op2, norm2, weight)
    zeros = jnp.zeros((ZR, DH), jnp.float32)
    partials = _sc_scatter(t, src, dst, zeros)
    return _merge(partials)
